# Initial kernel scaffold; baseline (speedup 1.0000x reference)
import jax
import jax.numpy as jnp
from jax.experimental import pallas as pl


def kernel(x, edge_index, W1, b1, W2, b2):
    raise NotImplementedError("write your pallas kernel here")


# speedup vs baseline: 4.2386x; 4.2386x over previous
import functools

import jax
import jax.numpy as jnp
from jax import lax
from jax.experimental import pallas as pl
from jax.experimental.pallas import tpu as pltpu
from jax.experimental.pallas import tpu_sc as plsc

N = 10000
E = 320000
D = 128

NC = 2
NS = 16
NW = NC * NS

EBLK = 80
E_PER_TILE = E // NW
NBLK = E_PER_TILE // EBLK
NPAD = 10240
ROWS_PER_TILE = NPAD // NS


@functools.cache
def _sc_kernels():
    mesh = plsc.VectorSubcoreMesh(
        core_axis_name="c", subcore_axis_name="s",
        num_cores=NC, num_subcores=NS)

    @functools.partial(
        pl.kernel,
        out_type=jax.ShapeDtypeStruct((NC, NPAD, D), jnp.float32),
        mesh=mesh,
        scratch_types=[
            pltpu.VMEM((EBLK,), jnp.int32),
            pltpu.VMEM((EBLK,), jnp.int32),
            pltpu.VMEM((EBLK, D), jnp.float32),
            pltpu.VMEM((EBLK, D), jnp.float32),
            pltpu.VMEM_SHARED((NPAD, D), jnp.float32),
        ],
    )
    def degree_pass(src_hbm, dst_hbm, ones_l_hbm, ones_r_hbm, zeros_hbm,
                    deg_hbm, idx_s, idx_d, ones_l, ones_r, acc):
        c = lax.axis_index("c")
        s = lax.axis_index("s")
        base = (c * NS + s) * E_PER_TILE
        r0 = s * ROWS_PER_TILE

        pltpu.sync_copy(ones_l_hbm, ones_l)
        pltpu.sync_copy(ones_r_hbm, ones_r)
        pltpu.sync_copy(zeros_hbm, acc.at[pl.ds(r0, ROWS_PER_TILE)])
        plsc.subcore_barrier()

        def body(i, carry):
            e0 = base + i * EBLK
            pltpu.sync_copy(src_hbm.at[pl.ds(e0, EBLK)], idx_s)
            pltpu.sync_copy(dst_hbm.at[pl.ds(e0, EBLK)], idx_d)
            pltpu.sync_copy(ones_l, acc.at[idx_s], add=True)
            pltpu.sync_copy(ones_r, acc.at[idx_d], add=True)
            return carry

        lax.fori_loop(0, NBLK, body, 0)
        plsc.subcore_barrier()

        pltpu.sync_copy(acc.at[pl.ds(r0, ROWS_PER_TILE)],
                        deg_hbm.at[c, pl.ds(r0, ROWS_PER_TILE)])

    @functools.partial(
        pl.kernel,
        out_type=jax.ShapeDtypeStruct((NC, NPAD, D), jnp.float32),
        mesh=mesh,
        scratch_types=[
            pltpu.VMEM((EBLK,), jnp.int32),
            pltpu.VMEM((EBLK,), jnp.int32),
            pltpu.VMEM((EBLK, D), jnp.float32),
            pltpu.VMEM_SHARED((NPAD, D), jnp.float32),
            pltpu.SemaphoreType.DMA,
        ],
    )
    def edge_pass(h_hbm, src_hbm, dst_hbm, zeros_hbm, out_hbm,
                  idx_s, idx_d, rows, acc, sem):
        c = lax.axis_index("c")
        s = lax.axis_index("s")
        base = (c * NS + s) * E_PER_TILE
        r0 = s * ROWS_PER_TILE

        pltpu.sync_copy(zeros_hbm, acc.at[pl.ds(r0, ROWS_PER_TILE)])
        plsc.subcore_barrier()

        def body(i, carry):
            e0 = base + i * EBLK
            pltpu.sync_copy(src_hbm.at[pl.ds(e0, EBLK)], idx_s)
            pltpu.sync_copy(dst_hbm.at[pl.ds(e0, EBLK)], idx_d)
            pltpu.async_copy(h_hbm.at[idx_s], rows, sem).wait()
            pltpu.sync_copy(rows, acc.at[idx_d], add=True)
            return carry

        lax.fori_loop(0, NBLK, body, 0)
        plsc.subcore_barrier()

        pltpu.sync_copy(acc.at[pl.ds(r0, ROWS_PER_TILE)],
                        out_hbm.at[c, pl.ds(r0, ROWS_PER_TILE)])

    return degree_pass, edge_pass


R = 400
GRID = N // R


def _norm(degp_blk, col):
    deg = degp_blk[0, :, col:col + 1] + degp_blk[1, :, col:col + 1]
    return jnp.where(deg > 0, lax.rsqrt(jnp.maximum(deg, 1e-12)), 0.0)


def _mm1_body(x_ref, degp_ref, w_ref, o_ref):
    h = x_ref[...] * _norm(degp_ref, 0)
    o_ref[...] = jnp.dot(h, w_ref[...], preferred_element_type=jnp.float32)


def _mm2_body(aggp_ref, degp_ref, b_ref, w_ref, o_ref):
    agg = aggp_ref[0] + aggp_ref[1]
    t = jnp.maximum(agg * _norm(degp_ref, 64) + b_ref[...], 0.0)
    h = t * _norm(degp_ref, 0)
    o_ref[...] = jnp.dot(h, w_ref[...], preferred_element_type=jnp.float32)


def _fin_body(aggp_ref, degp_ref, b_ref, o_ref):
    agg = aggp_ref[0] + aggp_ref[1]
    o_ref[...] = agg * _norm(degp_ref, 64) + b_ref[...]


_row_spec = pl.BlockSpec((R, D), lambda i: (i, 0))
_agg_spec = pl.BlockSpec((2, R, D), lambda i: (0, i, 0))
_b_spec = pl.BlockSpec((1, D), lambda i: (0, 0))
_w_spec = pl.BlockSpec((D, D), lambda i: (0, 0))

_mm1 = pl.pallas_call(
    _mm1_body,
    grid=(GRID,),
    in_specs=[_row_spec, _agg_spec, _w_spec],
    out_specs=_row_spec,
    out_shape=jax.ShapeDtypeStruct((N, D), jnp.float32),
)

_mm2 = pl.pallas_call(
    _mm2_body,
    grid=(GRID,),
    in_specs=[_agg_spec, _agg_spec, _b_spec, _w_spec],
    out_specs=_row_spec,
    out_shape=jax.ShapeDtypeStruct((N, D), jnp.float32),
)

_fin = pl.pallas_call(
    _fin_body,
    grid=(GRID,),
    in_specs=[_agg_spec, _agg_spec, _b_spec],
    out_specs=_row_spec,
    out_shape=jax.ShapeDtypeStruct((N, D), jnp.float32),
)


def kernel(x, edge_index, W1, b1, W2, b2):
    src = edge_index[0]
    dst = edge_index[1]
    ones_l = jnp.concatenate(
        [jnp.ones((EBLK, 64), jnp.float32), jnp.zeros((EBLK, 64), jnp.float32)], 1)
    ones_r = jnp.concatenate(
        [jnp.zeros((EBLK, 64), jnp.float32), jnp.ones((EBLK, 64), jnp.float32)], 1)
    zerosD = jnp.zeros((ROWS_PER_TILE, D), jnp.float32)
    b1r = b1.reshape(1, D)
    b2r = b2.reshape(1, D)

    degree_pass, edge_pass = _sc_kernels()
    deg_p = degree_pass(src, dst, ones_l, ones_r, zerosD)
    h1 = _mm1(x, deg_p, W1)
    agg1_p = edge_pass(h1, src, dst, zerosD)
    h2 = _mm2(agg1_p, deg_p, b1r, W2)
    agg2_p = edge_pass(h2, src, dst, zerosD)
    return _fin(agg2_p, deg_p, b2r)


# speedup vs baseline: 6.1313x; 1.4465x over previous
import functools

import jax
import jax.numpy as jnp
from jax import lax
from jax.experimental import pallas as pl
from jax.experimental.pallas import tpu as pltpu
from jax.experimental.pallas import tpu_sc as plsc

N = 10000
E = 320000
D = 128

NC = 2
NS = 16
NW = NC * NS

EBLK = 80
NBLK = 128
E_PAD = NW * NBLK * EBLK
NPAD = 10240
ROWS_PER_TILE = NPAD // NS
DUMMY = NPAD - 1


@functools.cache
def _sc_kernels():
    mesh = plsc.VectorSubcoreMesh(
        core_axis_name="c", subcore_axis_name="s",
        num_cores=NC, num_subcores=NS)

    @functools.partial(
        pl.kernel,
        out_type=jax.ShapeDtypeStruct((NC, NPAD, D), jnp.float32),
        mesh=mesh,
        scratch_types=[
            pltpu.VMEM((EBLK,), jnp.int32),
            pltpu.VMEM((EBLK,), jnp.int32),
            pltpu.VMEM((EBLK, D), jnp.float32),
            pltpu.VMEM((EBLK, D), jnp.float32),
            pltpu.VMEM_SHARED((NPAD, D), jnp.float32),
        ],
    )
    def degree_pass(src_hbm, dst_hbm, ones_l_hbm, ones_r_hbm, zeros_hbm,
                    deg_hbm, idx_s, idx_d, ones_l, ones_r, acc):
        c = lax.axis_index("c")
        s = lax.axis_index("s")
        base = (c * NS + s) * NBLK * EBLK
        r0 = s * ROWS_PER_TILE

        pltpu.sync_copy(ones_l_hbm, ones_l)
        pltpu.sync_copy(ones_r_hbm, ones_r)
        pltpu.sync_copy(zeros_hbm, acc.at[pl.ds(r0, ROWS_PER_TILE)])
        plsc.subcore_barrier()

        def body(i, carry):
            e0 = base + i * EBLK
            pltpu.sync_copy(src_hbm.at[pl.ds(e0, EBLK)], idx_s)
            pltpu.sync_copy(dst_hbm.at[pl.ds(e0, EBLK)], idx_d)
            pltpu.sync_copy(ones_l, acc.at[idx_s], add=True)
            pltpu.sync_copy(ones_r, acc.at[idx_d], add=True)
            return carry

        lax.fori_loop(0, NBLK, body, 0)
        plsc.subcore_barrier()

        pltpu.sync_copy(acc.at[pl.ds(r0, ROWS_PER_TILE)],
                        deg_hbm.at[c, pl.ds(r0, ROWS_PER_TILE)])

    @functools.partial(
        pl.kernel,
        out_type=jax.ShapeDtypeStruct((NC, NPAD, D), jnp.float32),
        mesh=mesh,
        scratch_types=[
            pltpu.VMEM((8, EBLK), jnp.int32),
            pltpu.VMEM((8, EBLK), jnp.int32),
            pltpu.VMEM((8, EBLK), jnp.int32),
            pltpu.VMEM((8, EBLK), jnp.int32),
            pltpu.VMEM((2, EBLK, D), jnp.float32),
            pltpu.VMEM_SHARED((NPAD, D), jnp.float32),
            pltpu.SemaphoreType.DMA,
            pltpu.SemaphoreType.DMA,
            pltpu.SemaphoreType.DMA,
            pltpu.SemaphoreType.DMA,
            pltpu.SemaphoreType.DMA,
            pltpu.SemaphoreType.DMA,
        ],
    )
    def edge_pass(h_hbm, src3_hbm, dst3_hbm, zeros_hbm, out_hbm,
                  isA, isB, idA, idB, rows, acc,
                  g0, g1, s0, s1, isemA, isemB):
        c = lax.axis_index("c")
        s = lax.axis_index("s")
        wid = c * NS + s
        r0 = s * ROWS_PER_TILE
        gsems = (g0, g1)
        ssems = (s0, s1)
        isbufs = (isA, isB)
        idbufs = (idA, idB)
        isems = (isemA, isemB)

        def idx_fire(p, sb):
            pltpu.async_copy(src3_hbm.at[wid, pl.ds(sb * 8, 8)],
                             isbufs[p], isems[p])
            pltpu.async_copy(dst3_hbm.at[wid, pl.ds(sb * 8, 8)],
                             idbufs[p], isems[p])

        def idx_wait(p):
            pltpu.make_async_copy(
                src3_hbm.at[wid, pl.ds(0, 8)], isbufs[p], isems[p]).wait()
            pltpu.make_async_copy(
                src3_hbm.at[wid, pl.ds(0, 8)], idbufs[p], isems[p]).wait()

        def g_start(j1, b, t_off):
            pltpu.async_copy(h_hbm.at[isbufs[(j1 // 8) % 2].at[j1 % 8]],
                             rows.at[b], gsems[b])

        def g_wait(b):
            pltpu.make_async_copy(
                h_hbm.at[pl.ds(0, EBLK)], rows.at[b], gsems[b]).wait()

        def s_start(j, b):
            pltpu.async_copy(rows.at[b],
                             acc.at[idbufs[j // 8].at[j % 8]],
                             ssems[b], add=True)

        def s_wait(b):
            pltpu.make_async_copy(
                h_hbm.at[pl.ds(0, EBLK)], rows.at[b], ssems[b]).wait()

        pltpu.sync_copy(src3_hbm.at[wid, pl.ds(0, 8)], isA)
        pltpu.sync_copy(dst3_hbm.at[wid, pl.ds(0, 8)], idA)
        pltpu.sync_copy(zeros_hbm, acc.at[pl.ds(r0, ROWS_PER_TILE)])
        plsc.subcore_barrier()
        g_start(0, 0, 0)

        def body(t, carry):
            for j in range(16):
                b = j % 2
                g_wait(b)
                s_start(j, b)
                if j == 0:
                    @pl.when(t > 0)
                    def _():
                        s_wait(1)
                else:
                    s_wait(1 - b)
                if j == 1:
                    idx_fire(1, 2 * t + 1)
                if j == 7:
                    idx_wait(1)
                if j == 9:
                    @pl.when(t < 7)
                    def _():
                        idx_fire(0, 2 * t + 2)
                if j < 15:
                    g_start(j + 1, 1 - b, t)
                else:
                    @pl.when(t < 7)
                    def _():
                        idx_wait(0)
                        g_start(0, 1 - b, t)
            return carry

        lax.fori_loop(0, 8, body, 0)
        s_wait(1)
        plsc.subcore_barrier()

        pltpu.sync_copy(acc.at[pl.ds(r0, ROWS_PER_TILE)],
                        out_hbm.at[c, pl.ds(r0, ROWS_PER_TILE)])

    return degree_pass, edge_pass


R = 512
GRID = NPAD // R


def _norm(degp_blk, col):
    deg = degp_blk[0, :, col:col + 1] + degp_blk[1, :, col:col + 1]
    return jnp.where(deg > 0, lax.rsqrt(jnp.maximum(deg, 1e-12)), 0.0)


def _pad_mask(i):
    row = i * R + lax.broadcasted_iota(jnp.int32, (R, 1), 0)
    return row < N


def _mm1_body(x_ref, degp_ref, w_ref, o_ref):
    h = x_ref[...] * _norm(degp_ref, 0)
    o_ref[...] = jnp.dot(h, w_ref[...], preferred_element_type=jnp.float32)


def _mm2_body(aggp_ref, degp_ref, b_ref, w_ref, o_ref):
    i = pl.program_id(0)
    agg = aggp_ref[0] + aggp_ref[1]
    t = jnp.maximum(agg * _norm(degp_ref, 64) + b_ref[...], 0.0)
    h = jnp.where(_pad_mask(i), t * _norm(degp_ref, 0), 0.0)
    o_ref[...] = jnp.dot(h, w_ref[...], preferred_element_type=jnp.float32)


def _fin_body(aggp_ref, degp_ref, b_ref, o_ref):
    agg = aggp_ref[0] + aggp_ref[1]
    o_ref[...] = agg * _norm(degp_ref, 64) + b_ref[...]


_row_spec = pl.BlockSpec((R, D), lambda i: (i, 0))
_agg_spec = pl.BlockSpec((2, R, D), lambda i: (0, i, 0))
_b_spec = pl.BlockSpec((1, D), lambda i: (0, 0))
_w_spec = pl.BlockSpec((D, D), lambda i: (0, 0))

_mm1 = pl.pallas_call(
    _mm1_body,
    grid=(GRID,),
    in_specs=[_row_spec, _agg_spec, _w_spec],
    out_specs=_row_spec,
    out_shape=jax.ShapeDtypeStruct((NPAD, D), jnp.float32),
)

_mm2 = pl.pallas_call(
    _mm2_body,
    grid=(GRID,),
    in_specs=[_agg_spec, _agg_spec, _b_spec, _w_spec],
    out_specs=_row_spec,
    out_shape=jax.ShapeDtypeStruct((NPAD, D), jnp.float32),
)

_fin = pl.pallas_call(
    _fin_body,
    grid=(GRID,),
    in_specs=[_agg_spec, _agg_spec, _b_spec],
    out_specs=_row_spec,
    out_shape=jax.ShapeDtypeStruct((NPAD, D), jnp.float32),
)


def kernel(x, edge_index, W1, b1, W2, b2):
    pad = N + jnp.arange(E_PAD - E, dtype=jnp.int32) % (NPAD - N)
    src3 = jnp.concatenate([edge_index[0], pad]).reshape(NW, NBLK, EBLK)
    dst3 = jnp.concatenate([edge_index[1], pad]).reshape(NW, NBLK, EBLK)
    xp = jnp.concatenate([x, jnp.zeros((NPAD - N, D), jnp.float32)], 0)
    ones_l = jnp.concatenate(
        [jnp.ones((EBLK, 64), jnp.float32), jnp.zeros((EBLK, 64), jnp.float32)], 1)
    ones_r = jnp.concatenate(
        [jnp.zeros((EBLK, 64), jnp.float32), jnp.ones((EBLK, 64), jnp.float32)], 1)
    zerosD = jnp.zeros((ROWS_PER_TILE, D), jnp.float32)
    b1r = b1.reshape(1, D)
    b2r = b2.reshape(1, D)

    src_flat = src3.reshape(E_PAD)
    dst_flat = dst3.reshape(E_PAD)
    degree_pass, edge_pass = _sc_kernels()
    deg_p = degree_pass(src_flat, dst_flat, ones_l, ones_r, zerosD)
    h1 = _mm1(xp, deg_p, W1)
    agg1_p = edge_pass(h1, src3, dst3, zerosD)
    h2 = _mm2(agg1_p, deg_p, b1r, W2)
    agg2_p = edge_pass(h2, src3, dst3, zerosD)
    return _fin(agg2_p, deg_p, b2r)[:N]


# speedup vs baseline: 7.3333x; 1.1961x over previous
import functools

import jax
import jax.numpy as jnp
from jax import lax
from jax.experimental import pallas as pl
from jax.experimental.pallas import tpu as pltpu
from jax.experimental.pallas import tpu_sc as plsc

N = 10000
E = 320000
D = 128

NC = 2
NS = 16
NW = NC * NS

EBLK = 80
NBLK = 128
E_PAD = NW * NBLK * EBLK
NPAD = 10240
ROWS_PER_TILE = NPAD // NS
DUMMY = NPAD - 1


@functools.cache
def _sc_kernels():
    mesh = plsc.VectorSubcoreMesh(
        core_axis_name="c", subcore_axis_name="s",
        num_cores=NC, num_subcores=NS)

    @functools.partial(
        pl.kernel,
        out_type=jax.ShapeDtypeStruct((NC, NPAD, D), jnp.float32),
        mesh=mesh,
        scratch_types=[
            pltpu.VMEM((8, EBLK), jnp.int32),
            pltpu.VMEM((8, EBLK), jnp.int32),
            pltpu.VMEM((8, EBLK), jnp.int32),
            pltpu.VMEM((8, EBLK), jnp.int32),
            pltpu.VMEM((EBLK, D), jnp.float32),
            pltpu.VMEM((EBLK, D), jnp.float32),
            pltpu.VMEM_SHARED((NPAD, D), jnp.float32),
            pltpu.SemaphoreType.DMA,
            pltpu.SemaphoreType.DMA,
            pltpu.SemaphoreType.DMA,
            pltpu.SemaphoreType.DMA,
        ],
    )
    def degree_pass(src3_hbm, dst3_hbm, ones_l_hbm, ones_r_hbm, zeros_hbm,
                    deg_hbm,
                    isA, isB, idA, idB, ones_l, ones_r, acc,
                    sA, sB, isemA, isemB):
        c = lax.axis_index("c")
        s = lax.axis_index("s")
        wid = c * NS + s
        r0 = s * ROWS_PER_TILE
        ssems = (sA, sB)
        isbufs = (isA, isB)
        idbufs = (idA, idB)
        isems = (isemA, isemB)

        def idx_fire(p, sb):
            pltpu.async_copy(src3_hbm.at[wid, pl.ds(sb * 8, 8)],
                             isbufs[p], isems[p])
            pltpu.async_copy(dst3_hbm.at[wid, pl.ds(sb * 8, 8)],
                             idbufs[p], isems[p])

        def idx_wait(p):
            pltpu.make_async_copy(
                src3_hbm.at[wid, pl.ds(0, 8)], isbufs[p], isems[p]).wait()
            pltpu.make_async_copy(
                src3_hbm.at[wid, pl.ds(0, 8)], idbufs[p], isems[p]).wait()

        def sc_fire(j):
            p, r = j // 8, j % 8
            pltpu.async_copy(ones_l, acc.at[isbufs[p].at[r]],
                             ssems[p], add=True)
            pltpu.async_copy(ones_r, acc.at[idbufs[p].at[r]],
                             ssems[p], add=True)

        def sc_drain(p):
            for _ in range(16):
                pltpu.make_async_copy(
                    zeros_hbm.at[pl.ds(0, EBLK)], ones_l, ssems[p]).wait()

        pltpu.sync_copy(ones_l_hbm, ones_l)
        pltpu.sync_copy(ones_r_hbm, ones_r)
        pltpu.sync_copy(src3_hbm.at[wid, pl.ds(0, 8)], isA)
        pltpu.sync_copy(dst3_hbm.at[wid, pl.ds(0, 8)], idA)
        pltpu.sync_copy(zeros_hbm, acc.at[pl.ds(r0, ROWS_PER_TILE)])
        plsc.subcore_barrier()

        def body(t, carry):
            for j in range(16):
                if j == 0:
                    @pl.when(t > 0)
                    def _():
                        sc_drain(1)
                if j == 1:
                    idx_fire(1, 2 * t + 1)
                if j == 7:
                    idx_wait(1)
                if j == 11:
                    @pl.when(t < 7)
                    def _():
                        sc_drain(0)
                        idx_fire(0, 2 * t + 2)
                if j == 14:
                    @pl.when(t < 7)
                    def _():
                        idx_wait(0)
                sc_fire(j)
            return carry

        lax.fori_loop(0, 8, body, 0)
        sc_drain(0)
        sc_drain(1)
        plsc.subcore_barrier()

        pltpu.sync_copy(acc.at[pl.ds(r0, ROWS_PER_TILE)],
                        deg_hbm.at[c, pl.ds(r0, ROWS_PER_TILE)])

    @functools.partial(
        pl.kernel,
        out_type=jax.ShapeDtypeStruct((NC, NPAD, D), jnp.float32),
        mesh=mesh,
        scratch_types=[
            pltpu.VMEM((8, EBLK), jnp.int32),
            pltpu.VMEM((8, EBLK), jnp.int32),
            pltpu.VMEM((8, EBLK), jnp.int32),
            pltpu.VMEM((8, EBLK), jnp.int32),
            pltpu.VMEM((2, EBLK, D), jnp.float32),
            pltpu.VMEM_SHARED((NPAD, D), jnp.float32),
            pltpu.SemaphoreType.DMA,
            pltpu.SemaphoreType.DMA,
            pltpu.SemaphoreType.DMA,
            pltpu.SemaphoreType.DMA,
            pltpu.SemaphoreType.DMA,
            pltpu.SemaphoreType.DMA,
        ],
    )
    def edge_pass(h_hbm, src3_hbm, dst3_hbm, zeros_hbm, out_hbm,
                  isA, isB, idA, idB, rows, acc,
                  g0, g1, s0, s1, isemA, isemB):
        c = lax.axis_index("c")
        s = lax.axis_index("s")
        wid = c * NS + s
        r0 = s * ROWS_PER_TILE
        gsems = (g0, g1)
        ssems = (s0, s1)
        isbufs = (isA, isB)
        idbufs = (idA, idB)
        isems = (isemA, isemB)

        def idx_fire(p, sb):
            pltpu.async_copy(src3_hbm.at[wid, pl.ds(sb * 8, 8)],
                             isbufs[p], isems[p])
            pltpu.async_copy(dst3_hbm.at[wid, pl.ds(sb * 8, 8)],
                             idbufs[p], isems[p])

        def idx_wait(p):
            pltpu.make_async_copy(
                src3_hbm.at[wid, pl.ds(0, 8)], isbufs[p], isems[p]).wait()
            pltpu.make_async_copy(
                src3_hbm.at[wid, pl.ds(0, 8)], idbufs[p], isems[p]).wait()

        def g_start(j1, b, t_off):
            pltpu.async_copy(h_hbm.at[isbufs[(j1 // 8) % 2].at[j1 % 8]],
                             rows.at[b], gsems[b])

        def g_wait(b):
            pltpu.make_async_copy(
                h_hbm.at[pl.ds(0, EBLK)], rows.at[b], gsems[b]).wait()

        def s_start(j, b):
            pltpu.async_copy(rows.at[b],
                             acc.at[idbufs[j // 8].at[j % 8]],
                             ssems[b], add=True)

        def s_wait(b):
            pltpu.make_async_copy(
                h_hbm.at[pl.ds(0, EBLK)], rows.at[b], ssems[b]).wait()

        pltpu.sync_copy(src3_hbm.at[wid, pl.ds(0, 8)], isA)
        pltpu.sync_copy(dst3_hbm.at[wid, pl.ds(0, 8)], idA)
        pltpu.sync_copy(zeros_hbm, acc.at[pl.ds(r0, ROWS_PER_TILE)])
        plsc.subcore_barrier()
        g_start(0, 0, 0)

        def body(t, carry):
            for j in range(16):
                b = j % 2
                g_wait(b)
                s_start(j, b)
                if j == 0:
                    @pl.when(t > 0)
                    def _():
                        s_wait(1)
                else:
                    s_wait(1 - b)
                if j == 1:
                    idx_fire(1, 2 * t + 1)
                if j == 7:
                    idx_wait(1)
                if j == 9:
                    @pl.when(t < 7)
                    def _():
                        idx_fire(0, 2 * t + 2)
                if j < 15:
                    g_start(j + 1, 1 - b, t)
                else:
                    @pl.when(t < 7)
                    def _():
                        idx_wait(0)
                        g_start(0, 1 - b, t)
            return carry

        lax.fori_loop(0, 8, body, 0)
        s_wait(1)
        plsc.subcore_barrier()

        pltpu.sync_copy(acc.at[pl.ds(r0, ROWS_PER_TILE)],
                        out_hbm.at[c, pl.ds(r0, ROWS_PER_TILE)])

    return degree_pass, edge_pass


R = 512
GRID = NPAD // R


def _norm(degp_blk, col):
    deg = degp_blk[0, :, col:col + 1] + degp_blk[1, :, col:col + 1]
    return jnp.where(deg > 0, lax.rsqrt(jnp.maximum(deg, 1e-12)), 0.0)


def _pad_mask(i):
    row = i * R + lax.broadcasted_iota(jnp.int32, (R, 1), 0)
    return row < N


def _mm1_body(x_ref, degp_ref, w_ref, o_ref):
    h = x_ref[...] * _norm(degp_ref, 0)
    o_ref[...] = jnp.dot(h, w_ref[...], preferred_element_type=jnp.float32)


def _mm2_body(aggp_ref, degp_ref, b_ref, w_ref, o_ref):
    i = pl.program_id(0)
    agg = aggp_ref[0] + aggp_ref[1]
    t = jnp.maximum(agg * _norm(degp_ref, 64) + b_ref[...], 0.0)
    h = jnp.where(_pad_mask(i), t * _norm(degp_ref, 0), 0.0)
    o_ref[...] = jnp.dot(h, w_ref[...], preferred_element_type=jnp.float32)


def _fin_body(aggp_ref, degp_ref, b_ref, o_ref):
    agg = aggp_ref[0] + aggp_ref[1]
    o_ref[...] = agg * _norm(degp_ref, 64) + b_ref[...]


_row_spec = pl.BlockSpec((R, D), lambda i: (i, 0))
_agg_spec = pl.BlockSpec((2, R, D), lambda i: (0, i, 0))
_b_spec = pl.BlockSpec((1, D), lambda i: (0, 0))
_w_spec = pl.BlockSpec((D, D), lambda i: (0, 0))

_mm1 = pl.pallas_call(
    _mm1_body,
    grid=(GRID,),
    in_specs=[_row_spec, _agg_spec, _w_spec],
    out_specs=_row_spec,
    out_shape=jax.ShapeDtypeStruct((NPAD, D), jnp.float32),
)

_mm2 = pl.pallas_call(
    _mm2_body,
    grid=(GRID,),
    in_specs=[_agg_spec, _agg_spec, _b_spec, _w_spec],
    out_specs=_row_spec,
    out_shape=jax.ShapeDtypeStruct((NPAD, D), jnp.float32),
)

_fin = pl.pallas_call(
    _fin_body,
    grid=(GRID,),
    in_specs=[_agg_spec, _agg_spec, _b_spec],
    out_specs=_row_spec,
    out_shape=jax.ShapeDtypeStruct((NPAD, D), jnp.float32),
)


def kernel(x, edge_index, W1, b1, W2, b2):
    pad = N + jnp.arange(E_PAD - E, dtype=jnp.int32) % (NPAD - N)
    src3 = jnp.concatenate([edge_index[0], pad]).reshape(NW, NBLK, EBLK)
    dst3 = jnp.concatenate([edge_index[1], pad]).reshape(NW, NBLK, EBLK)
    xp = jnp.concatenate([x, jnp.zeros((NPAD - N, D), jnp.float32)], 0)
    ones_l = jnp.concatenate(
        [jnp.ones((EBLK, 64), jnp.float32), jnp.zeros((EBLK, 64), jnp.float32)], 1)
    ones_r = jnp.concatenate(
        [jnp.zeros((EBLK, 64), jnp.float32), jnp.ones((EBLK, 64), jnp.float32)], 1)
    zerosD = jnp.zeros((ROWS_PER_TILE, D), jnp.float32)
    b1r = b1.reshape(1, D)
    b2r = b2.reshape(1, D)

    degree_pass, edge_pass = _sc_kernels()
    deg_p = degree_pass(src3, dst3, ones_l, ones_r, zerosD)
    h1 = _mm1(xp, deg_p, W1)
    agg1_p = edge_pass(h1, src3, dst3, zerosD)
    h2 = _mm2(agg1_p, deg_p, b1r, W2)
    agg2_p = edge_pass(h2, src3, dst3, zerosD)
    return _fin(agg2_p, deg_p, b2r)[:N]


# speedup vs baseline: 9.1582x; 1.2488x over previous
import functools

import jax
import jax.numpy as jnp
from jax import lax
from jax.experimental import pallas as pl
from jax.experimental.pallas import tpu as pltpu
from jax.experimental.pallas import tpu_sc as plsc

N = 10000
E = 320000
D = 128

NC = 2
NS = 16
NW = NC * NS

EBLK = 80
NBLK = 128
E_PAD = NW * NBLK * EBLK
NPAD = 10240
ROWS_PER_TILE = NPAD // NS
DUMMY = NPAD - 1


@functools.cache
def _sc_kernels():
    mesh = plsc.VectorSubcoreMesh(
        core_axis_name="c", subcore_axis_name="s",
        num_cores=NC, num_subcores=NS)

    @functools.partial(
        pl.kernel,
        out_type=jax.ShapeDtypeStruct((NC, NPAD, D), jnp.float32),
        mesh=mesh,
        scratch_types=[
            pltpu.VMEM((8, EBLK), jnp.int32),
            pltpu.VMEM((8, EBLK), jnp.int32),
            pltpu.VMEM((8, EBLK), jnp.int32),
            pltpu.VMEM((8, EBLK), jnp.int32),
            pltpu.VMEM((EBLK, D), jnp.float32),
            pltpu.VMEM((EBLK, D), jnp.float32),
            pltpu.VMEM_SHARED((NPAD, D), jnp.float32),
            pltpu.SemaphoreType.DMA,
            pltpu.SemaphoreType.DMA,
            pltpu.SemaphoreType.DMA,
            pltpu.SemaphoreType.DMA,
        ],
    )
    def degree_pass(src3_hbm, dst3_hbm, ones_l_hbm, ones_r_hbm, zeros_hbm,
                    deg_hbm,
                    isA, isB, idA, idB, ones_l, ones_r, acc,
                    sA, sB, isemA, isemB):
        c = lax.axis_index("c")
        s = lax.axis_index("s")
        wid = c * NS + s
        r0 = s * ROWS_PER_TILE
        ssems = (sA, sB)
        isbufs = (isA, isB)
        idbufs = (idA, idB)
        isems = (isemA, isemB)

        def idx_fire(p, sb):
            pltpu.async_copy(src3_hbm.at[wid, pl.ds(sb * 8, 8)],
                             isbufs[p], isems[p])
            pltpu.async_copy(dst3_hbm.at[wid, pl.ds(sb * 8, 8)],
                             idbufs[p], isems[p])

        def idx_wait(p):
            pltpu.make_async_copy(
                src3_hbm.at[wid, pl.ds(0, 8)], isbufs[p], isems[p]).wait()
            pltpu.make_async_copy(
                src3_hbm.at[wid, pl.ds(0, 8)], idbufs[p], isems[p]).wait()

        def sc_fire(j):
            p, r = j // 8, j % 8
            pltpu.async_copy(ones_l, acc.at[isbufs[p].at[r]],
                             ssems[p], add=True)
            pltpu.async_copy(ones_r, acc.at[idbufs[p].at[r]],
                             ssems[p], add=True)

        def sc_drain(p):
            for _ in range(16):
                pltpu.make_async_copy(
                    zeros_hbm.at[pl.ds(0, EBLK)], ones_l, ssems[p]).wait()

        pltpu.sync_copy(ones_l_hbm, ones_l)
        pltpu.sync_copy(ones_r_hbm, ones_r)
        pltpu.sync_copy(src3_hbm.at[wid, pl.ds(0, 8)], isA)
        pltpu.sync_copy(dst3_hbm.at[wid, pl.ds(0, 8)], idA)
        pltpu.sync_copy(zeros_hbm, acc.at[pl.ds(r0, ROWS_PER_TILE)])
        plsc.subcore_barrier()

        def body(t, carry):
            for j in range(16):
                if j == 0:
                    @pl.when(t > 0)
                    def _():
                        sc_drain(1)
                if j == 1:
                    idx_fire(1, 2 * t + 1)
                if j == 7:
                    idx_wait(1)
                if j == 11:
                    @pl.when(t < 7)
                    def _():
                        sc_drain(0)
                        idx_fire(0, 2 * t + 2)
                if j == 14:
                    @pl.when(t < 7)
                    def _():
                        idx_wait(0)
                sc_fire(j)
            return carry

        lax.fori_loop(0, 8, body, 0)
        sc_drain(0)
        sc_drain(1)
        plsc.subcore_barrier()

        pltpu.sync_copy(acc.at[pl.ds(r0, ROWS_PER_TILE)],
                        deg_hbm.at[c, pl.ds(r0, ROWS_PER_TILE)])

    @functools.partial(
        pl.kernel,
        out_type=jax.ShapeDtypeStruct((NC, NPAD, D), jnp.float32),
        mesh=mesh,
        scratch_types=[
            pltpu.VMEM((8, EBLK), jnp.int32),
            pltpu.VMEM((8, EBLK), jnp.int32),
            pltpu.VMEM((8, EBLK), jnp.int32),
            pltpu.VMEM((8, EBLK), jnp.int32),
            pltpu.VMEM((4, EBLK, D), jnp.float32),
            pltpu.VMEM_SHARED((NPAD, D), jnp.float32),
            pltpu.SemaphoreType.DMA,
            pltpu.SemaphoreType.DMA,
            pltpu.SemaphoreType.DMA,
            pltpu.SemaphoreType.DMA,
            pltpu.SemaphoreType.DMA,
            pltpu.SemaphoreType.DMA,
            pltpu.SemaphoreType.DMA,
            pltpu.SemaphoreType.DMA,
            pltpu.SemaphoreType.DMA,
            pltpu.SemaphoreType.DMA,
        ],
    )
    def edge_pass(h_hbm, src3_hbm, dst3_hbm, zeros_hbm, out_hbm,
                  isA, isB, idA, idB, rows, acc,
                  g0, g1, g2, g3, s0, s1, s2, s3, isemA, isemB):
        c = lax.axis_index("c")
        s = lax.axis_index("s")
        wid = c * NS + s
        r0 = s * ROWS_PER_TILE
        gsems = (g0, g1, g2, g3)
        ssems = (s0, s1, s2, s3)
        isbufs = (isA, isB)
        idbufs = (idA, idB)
        isems = (isemA, isemB)

        def idx_fire(p, sb):
            pltpu.async_copy(src3_hbm.at[wid, pl.ds(sb * 8, 8)],
                             isbufs[p], isems[p])
            pltpu.async_copy(dst3_hbm.at[wid, pl.ds(sb * 8, 8)],
                             idbufs[p], isems[p])

        def idx_wait(p):
            pltpu.make_async_copy(
                src3_hbm.at[wid, pl.ds(0, 8)], isbufs[p], isems[p]).wait()
            pltpu.make_async_copy(
                src3_hbm.at[wid, pl.ds(0, 8)], idbufs[p], isems[p]).wait()

        def g_start(j1, b, t_off):
            pltpu.async_copy(h_hbm.at[isbufs[(j1 // 8) % 2].at[j1 % 8]],
                             rows.at[b], gsems[b])

        def g_wait(b):
            pltpu.make_async_copy(
                h_hbm.at[pl.ds(0, EBLK)], rows.at[b], gsems[b]).wait()

        def s_start(j, b):
            pltpu.async_copy(rows.at[b],
                             acc.at[idbufs[j // 8].at[j % 8]],
                             ssems[b], add=True)

        def s_wait(b):
            pltpu.make_async_copy(
                h_hbm.at[pl.ds(0, EBLK)], rows.at[b], ssems[b]).wait()

        pltpu.sync_copy(src3_hbm.at[wid, pl.ds(0, 8)], isA)
        pltpu.sync_copy(dst3_hbm.at[wid, pl.ds(0, 8)], idA)
        pltpu.sync_copy(zeros_hbm, acc.at[pl.ds(r0, ROWS_PER_TILE)])
        plsc.subcore_barrier()
        g_start(0, 0, 0)
        g_start(1, 1, 0)
        g_start(2, 2, 0)

        def body(t, carry):
            for j in range(16):
                b = j % 4
                b3 = (j + 3) % 4
                g_wait(b)
                s_start(j, b)
                if j == 0:
                    @pl.when(t > 0)
                    def _():
                        s_wait(b3)
                else:
                    s_wait(b3)
                if j == 1:
                    idx_fire(1, 2 * t + 1)
                if j == 5:
                    idx_wait(1)
                if j == 9:
                    @pl.when(t < 7)
                    def _():
                        idx_fire(0, 2 * t + 2)
                if j <= 12:
                    g_start(j + 3, b3, t)
                elif j == 13:
                    @pl.when(t < 7)
                    def _():
                        idx_wait(0)
                        g_start(0, b3, t)
                else:
                    @pl.when(t < 7)
                    def _(b3=b3, j=j):
                        g_start(j + 3 - 16, b3, t)
            return carry

        lax.fori_loop(0, 8, body, 0)
        s_wait(3)
        plsc.subcore_barrier()

        pltpu.sync_copy(acc.at[pl.ds(r0, ROWS_PER_TILE)],
                        out_hbm.at[c, pl.ds(r0, ROWS_PER_TILE)])

    return degree_pass, edge_pass


R = 512
GRID = NPAD // R


def _norm(degp_blk, col):
    deg = degp_blk[0, :, col:col + 1] + degp_blk[1, :, col:col + 1]
    return jnp.where(deg > 0, lax.rsqrt(jnp.maximum(deg, 1e-12)), 0.0)


def _pad_mask(i):
    row = i * R + lax.broadcasted_iota(jnp.int32, (R, 1), 0)
    return row < N


def _mm1_body(x_ref, degp_ref, w_ref, o_ref):
    h = x_ref[...] * _norm(degp_ref, 0)
    o_ref[...] = jnp.dot(h, w_ref[...], preferred_element_type=jnp.float32)


def _mm2_body(aggp_ref, degp_ref, b_ref, w_ref, o_ref):
    i = pl.program_id(0)
    agg = aggp_ref[0] + aggp_ref[1]
    t = jnp.maximum(agg * _norm(degp_ref, 64) + b_ref[...], 0.0)
    h = jnp.where(_pad_mask(i), t * _norm(degp_ref, 0), 0.0)
    o_ref[...] = jnp.dot(h, w_ref[...], preferred_element_type=jnp.float32)


def _fin_body(aggp_ref, degp_ref, b_ref, o_ref):
    agg = aggp_ref[0] + aggp_ref[1]
    o_ref[...] = agg * _norm(degp_ref, 64) + b_ref[...]


_row_spec = pl.BlockSpec((R, D), lambda i: (i, 0))
_agg_spec = pl.BlockSpec((2, R, D), lambda i: (0, i, 0))
_b_spec = pl.BlockSpec((1, D), lambda i: (0, 0))
_w_spec = pl.BlockSpec((D, D), lambda i: (0, 0))

_mm1 = pl.pallas_call(
    _mm1_body,
    grid=(GRID,),
    in_specs=[_row_spec, _agg_spec, _w_spec],
    out_specs=_row_spec,
    out_shape=jax.ShapeDtypeStruct((NPAD, D), jnp.float32),
)

_mm2 = pl.pallas_call(
    _mm2_body,
    grid=(GRID,),
    in_specs=[_agg_spec, _agg_spec, _b_spec, _w_spec],
    out_specs=_row_spec,
    out_shape=jax.ShapeDtypeStruct((NPAD, D), jnp.float32),
)

_fin = pl.pallas_call(
    _fin_body,
    grid=(GRID,),
    in_specs=[_agg_spec, _agg_spec, _b_spec],
    out_specs=_row_spec,
    out_shape=jax.ShapeDtypeStruct((NPAD, D), jnp.float32),
)


def kernel(x, edge_index, W1, b1, W2, b2):
    pad = N + jnp.arange(E_PAD - E, dtype=jnp.int32) % (NPAD - N)
    src3 = jnp.concatenate([edge_index[0], pad]).reshape(NW, NBLK, EBLK)
    dst3 = jnp.concatenate([edge_index[1], pad]).reshape(NW, NBLK, EBLK)
    xp = jnp.concatenate([x, jnp.zeros((NPAD - N, D), jnp.float32)], 0)
    ones_l = jnp.concatenate(
        [jnp.ones((EBLK, 64), jnp.float32), jnp.zeros((EBLK, 64), jnp.float32)], 1)
    ones_r = jnp.concatenate(
        [jnp.zeros((EBLK, 64), jnp.float32), jnp.ones((EBLK, 64), jnp.float32)], 1)
    zerosD = jnp.zeros((ROWS_PER_TILE, D), jnp.float32)
    b1r = b1.reshape(1, D)
    b2r = b2.reshape(1, D)

    degree_pass, edge_pass = _sc_kernels()
    deg_p = degree_pass(src3, dst3, ones_l, ones_r, zerosD)
    h1 = _mm1(xp, deg_p, W1)
    agg1_p = edge_pass(h1, src3, dst3, zerosD)
    h2 = _mm2(agg1_p, deg_p, b1r, W2)
    agg2_p = edge_pass(h2, src3, dst3, zerosD)
    return _fin(agg2_p, deg_p, b2r)[:N]
